# Initial kernel scaffold; baseline (speedup 1.0000x reference)
import jax
import jax.numpy as jnp
from jax.experimental import pallas as pl


def kernel(positions, x, edge_attr, edge_index, batch, W1_0, W1_1, W1_2, W2_0, W2_1, W2_2):
    raise NotImplementedError("write your pallas kernel here")


# speedup vs baseline: 4.7267x; 4.7267x over previous
import functools

import jax
import jax.numpy as jnp
import numpy as np
from jax import lax
from jax.experimental import pallas as pl
from jax.experimental.pallas import tpu as pltpu
from jax.experimental.pallas import tpu_sc as plsc

N = 10000
E = 160000
G = 64
NB = 4
NA = 128

NCORE = 2
NSUB = 16
BLK = 64
NBLKS = E // BLK
ROW = 112
XWC = 256
S3 = float(np.sqrt(3.0))
S5 = float(np.sqrt(5.0))


def _k1_xw(x, w1p):
    bn = 2000

    def body(x_ref, w_ref, o_ref):
        o_ref[...] = jnp.dot(
            x_ref[...], w_ref[0], preferred_element_type=jnp.float32
        )[None]

    return pl.pallas_call(
        body,
        grid=(NCORE, N // bn),
        in_specs=[
            pl.BlockSpec((bn, NA), lambda c, j: (j, 0)),
            pl.BlockSpec((1, NA, XWC), lambda c, j: (c, 0, 0)),
        ],
        out_specs=pl.BlockSpec((1, bn, XWC), lambda c, j: (c, j, 0)),
        out_shape=jax.ShapeDtypeStruct((NCORE, N, XWC), jnp.float32),
    )(x, w1p)


def _k3_reduce(partials):
    def body(p_ref, o_ref):
        acc = jnp.zeros((G, 16), jnp.float32)
        for c in range(NCORE):
            for s in range(NSUB):
                acc = acc + p_ref[c, s]
        o_ref[...] = jnp.sum(acc, axis=1, keepdims=True)

    return pl.pallas_call(
        body,
        out_shape=jax.ShapeDtypeStruct((G, 1), jnp.float32),
    )(partials)


def _sc_kernel(xwa, xwb, pos, batch, src_r, dst_r, ea, w2t):
    mesh = plsc.VectorSubcoreMesh(
        core_axis_name="c", subcore_axis_name="s", num_cores=NCORE,
        num_subcores=NSUB,
    )

    @functools.partial(
        pl.kernel,
        out_type=jax.ShapeDtypeStruct((NCORE, NSUB, G * 16), jnp.float32),
        mesh=mesh,
        compiler_params=pltpu.CompilerParams(
            needs_layout_passes=False, use_tc_tiling_on_sc=False),
        scratch_types=[
            pltpu.VMEM_SHARED((N, ROW), jnp.float32),
            pltpu.VMEM((N,), jnp.int32),
            pltpu.VMEM((NB, 64), jnp.float32),
            pltpu.VMEM((BLK,), jnp.int32),
            pltpu.VMEM((BLK,), jnp.int32),
            pltpu.VMEM((BLK * NB,), jnp.float32),
            pltpu.VMEM((BLK, XWC), jnp.float32),
            pltpu.VMEM((BLK, ROW), jnp.float32),
            pltpu.VMEM((8 * BLK,), jnp.float32),
            pltpu.VMEM((G * 16,), jnp.float32),
            pltpu.VMEM((BLK, 16), jnp.float32),
            pltpu.VMEM((BLK, 16), jnp.float32),
            pltpu.SemaphoreType.DMA((8,)),
        ],
    )
    def body(xwa_h, xwb_h, pos_h, batch_h, src_h, dst_h, ea_h, w2t_h, out_h,
             nf_sh, batch_v, w2_v, src_v, dst_v, ea_v, xw_v, em_v,
             sh_v, gacc_v, ps_v, pd_v, sem):
        cid = lax.axis_index("c")
        sid = lax.axis_index("s")

        pltpu.sync_copy(batch_h, batch_v)
        pltpu.sync_copy(w2t_h.at[cid], w2_v)

        zero16 = jnp.zeros((16,), jnp.float32)
        ci32 = jnp.zeros((16,), jnp.int32)
        iota16 = lax.iota(jnp.int32, 16)

        def zrow(i, _):
            for o in range(0, ROW, 16):
                em_v[i, pl.ds(o, 16)] = zero16
            return 0

        lax.fori_loop(0, BLK, zrow, 0)

        for o in range(0, G * 16, 16):
            gacc_v[pl.ds(o, 16)] = zero16

        rows_per = N // NSUB
        for ch in range(9):
            pltpu.sync_copy(
                em_v.at[pl.ds(0, 64)],
                nf_sh.at[pl.ds(sid * rows_per + ch * 64, 64)],
            )
        pltpu.sync_copy(
            em_v.at[pl.ds(0, 49)],
            nf_sh.at[pl.ds(sid * rows_per + 576, 49)],
        )
        plsc.subcore_barrier()

        nblk = jnp.where(sid < NBLKS - (NBLKS // NSUB) * NSUB, 1, 0) + (
            NBLKS // NSUB
        )

        def bcast(ref, idx):
            return plsc.load_gather(ref, [ci32 + idx])

        def sh_pass(want1, want2):
            for g in range(BLK // 16):
                r16 = iota16 + g * 16
                exv = (plsc.load_gather(ps_v, [r16, ci32])
                       - plsc.load_gather(pd_v, [r16, ci32]))
                eyv = (plsc.load_gather(ps_v, [r16, ci32 + 1])
                       - plsc.load_gather(pd_v, [r16, ci32 + 1]))
                ezv = (plsc.load_gather(ps_v, [r16, ci32 + 2])
                       - plsc.load_gather(pd_v, [r16, ci32 + 2]))
                if want1:
                    sh_v[pl.ds(0 * BLK + g * 16, 16)] = S3 * exv
                    sh_v[pl.ds(1 * BLK + g * 16, 16)] = S3 * eyv
                    sh_v[pl.ds(2 * BLK + g * 16, 16)] = S3 * ezv
                if want2:
                    sh_v[pl.ds(3 * BLK + g * 16, 16)] = (S5 * S3) * (exv * ezv)
                    sh_v[pl.ds(4 * BLK + g * 16, 16)] = (S5 * S3) * (exv * eyv)
                    sh_v[pl.ds(5 * BLK + g * 16, 16)] = S5 * (
                        eyv * eyv - 0.5 * (exv * exv + ezv * ezv)
                    )
                    sh_v[pl.ds(6 * BLK + g * 16, 16)] = (S5 * S3) * (eyv * ezv)
                    sh_v[pl.ds(7 * BLK + g * 16, 16)] = (S5 * 0.5 * S3) * (
                        ezv * ezv - exv * exv
                    )

        def ea4(i):
            return [bcast(ea_v, i * NB + v) for v in range(NB)]

        def q4(i):
            e = ea4(i)
            q = []
            for j in range(4):
                acc = xw_v[i, pl.ds(0 * 64 + j * 16, 16)] * e[0]
                acc += xw_v[i, pl.ds(1 * 64 + j * 16, 16)] * e[1]
                acc += xw_v[i, pl.ds(2 * 64 + j * 16, 16)] * e[2]
                acc += xw_v[i, pl.ds(3 * 64 + j * 16, 16)] * e[3]
                q.append(acc)
            return q

        def blk_dma(b, stage_a):
            pltpu.async_copy(src_h.at[b], src_v, sem.at[0])
            pltpu.async_copy(dst_h.at[b], dst_v, sem.at[1])
            pltpu.async_copy(ea_h.at[b], ea_v, sem.at[2])
            pltpu.make_async_copy(src_h.at[b], src_v, sem.at[0]).wait()
            if stage_a:
                @pl.when(cid == 0)
                def _():
                    pltpu.async_copy(xwa_h.at[src_v], xw_v, sem.at[3])

                @pl.when(cid == 1)
                def _():
                    pltpu.async_copy(xwb_h.at[src_v], xw_v, sem.at[3])
            else:
                pltpu.async_copy(nf_sh.at[src_v], em_v, sem.at[3])
            pltpu.make_async_copy(dst_h.at[b], dst_v, sem.at[1]).wait()
            pltpu.async_copy(pos_h.at[src_v], ps_v, sem.at[4])
            pltpu.async_copy(pos_h.at[dst_v], pd_v, sem.at[5])
            pltpu.make_async_copy(pos_h.at[src_v], ps_v, sem.at[4]).wait()
            pltpu.make_async_copy(pos_h.at[dst_v], pd_v, sem.at[5]).wait()
            pltpu.make_async_copy(ea_h.at[b], ea_v, sem.at[2]).wait()

        def row_wait(b, stage_a):
            if stage_a:
                @pl.when(cid == 0)
                def _():
                    pltpu.make_async_copy(
                        xwa_h.at[src_v], xw_v, sem.at[3]).wait()

                @pl.when(cid == 1)
                def _():
                    pltpu.make_async_copy(
                        xwb_h.at[src_v], xw_v, sem.at[3]).wait()
            else:
                pltpu.make_async_copy(
                    nf_sh.at[src_v], em_v, sem.at[3]).wait()

        def stage_a(k, _):
            b = sid + k * NSUB
            blk_dma(b, True)

            @pl.when(cid == 0)
            def _():
                sh_pass(True, False)
                row_wait(b, True)

                def edge(i, _):
                    q = q4(i)
                    em_v[i, pl.ds(0, 16)] = q[0]
                    em_v[i, pl.ds(16, 16)] = q[1]
                    for m in range(3):
                        sm = bcast(sh_v, m * BLK + i)
                        em_v[i, pl.ds(32 + 24 * m, 16)] = q[2] * sm
                        em_v[i, pl.ds(48 + 24 * m, 16)] = q[3] * sm
                    return 0

                lax.fori_loop(0, BLK, edge, 0)

            @pl.when(cid == 1)
            def _():
                sh_pass(False, True)
                row_wait(b, True)

                def edge(i, _):
                    q = q4(i)
                    em_v[i, pl.ds(0, 16)] = q[0]
                    em_v[i, pl.ds(16, 16)] = q[1]
                    for m in range(5):
                        sm = bcast(sh_v, (3 + m) * BLK + i)
                        em_v[i, pl.ds(32 + 16 * m, 16)] = q[2] * sm
                    return 0

                lax.fori_loop(0, BLK, edge, 0)

            pltpu.sync_copy(em_v, nf_sh.at[dst_v], add=True)
            return 0

        lax.fori_loop(0, nblk, stage_a, 0)
        plsc.subcore_barrier()

        def a4(i):
            e = ea4(i)
            a = []
            for j in range(4):
                acc = w2_v[0, pl.ds(j * 16, 16)] * e[0]
                acc += w2_v[1, pl.ds(j * 16, 16)] * e[1]
                acc += w2_v[2, pl.ds(j * 16, 16)] * e[2]
                acc += w2_v[3, pl.ds(j * 16, 16)] * e[3]
                a.append(acc)
            return a

        def gupd(i, acc):
            d16 = bcast(dst_v, i)
            g16 = plsc.load_gather(batch_v, [d16])
            plsc.addupdate_scatter(gacc_v, [g16 * 16 + iota16], acc)

        def stage_b(k, _):
            b = sid + k * NSUB
            blk_dma(b, False)

            @pl.when(cid == 0)
            def _():
                sh_pass(True, False)
                row_wait(b, False)

                def edge(i, _):
                    a = a4(i)
                    acc = em_v[i, pl.ds(0, 16)] * a[0]
                    acc += em_v[i, pl.ds(16, 16)] * a[1]
                    for m in range(3):
                        sm = bcast(sh_v, m * BLK + i)
                        acc += (em_v[i, pl.ds(32 + 24 * m, 16)] * a[2]) * sm
                        acc += (em_v[i, pl.ds(48 + 24 * m, 16)] * a[3]) * sm
                    gupd(i, acc)
                    return 0

                lax.fori_loop(0, BLK, edge, 0)

            @pl.when(cid == 1)
            def _():
                sh_pass(False, True)
                row_wait(b, False)

                def edge(i, _):
                    a = a4(i)
                    acc = em_v[i, pl.ds(0, 16)] * a[0]
                    acc += em_v[i, pl.ds(16, 16)] * a[1]
                    for m in range(5):
                        sm = bcast(sh_v, (3 + m) * BLK + i)
                        acc += (em_v[i, pl.ds(32 + 16 * m, 16)] * a[2]) * sm
                    gupd(i, acc)
                    return 0

                lax.fori_loop(0, BLK, edge, 0)

            return 0

        lax.fori_loop(0, nblk, stage_b, 0)
        pltpu.sync_copy(gacc_v, out_h.at[cid, sid])

    return body(xwa, xwb, pos, batch, src_r, dst_r, ea, w2t)


def kernel(positions, x, edge_attr, edge_index, batch, W1_0, W1_1, W1_2,
           W2_0, W2_1, W2_2):
    inv1 = 1.0 / float(np.sqrt(NA * NB))
    fan2 = 416.0
    k0 = 1.0 / float(np.sqrt(fan2))
    k1 = 1.0 / float(np.sqrt(fan2 * 3.0))
    k2 = 1.0 / float(np.sqrt(fan2 * 5.0))

    w1cat = jnp.concatenate([W1_0, W1_1, W1_2], axis=2) * inv1
    zpad8 = jnp.zeros((NA, 8), jnp.float32)
    zpad16 = jnp.zeros((NA, 16), jnp.float32)
    segs0 = [jnp.concatenate(
        [w1cat[:, v, 0:32], w1cat[:, v, 64:88], zpad8], axis=1)
        for v in range(NB)]
    segs1 = [jnp.concatenate(
        [w1cat[:, v, 32:64], w1cat[:, v, 88:104], zpad16], axis=1)
        for v in range(NB)]
    w1p0 = jnp.concatenate(segs0, axis=1)
    w1p1 = jnp.concatenate(segs1, axis=1)
    w1p = jnp.stack([w1p0, w1p1])

    z8 = jnp.zeros((NB, 8), jnp.float32)
    z16 = jnp.zeros((NB, 16), jnp.float32)
    w2t0 = jnp.concatenate(
        [W2_0[0:32, :, 0].T * k0, W2_1[:, :, 0].T * k1, z8], axis=1)
    w2t1 = jnp.concatenate(
        [W2_0[32:64, :, 0].T * k0, W2_2[:, :, 0].T * k2, z16], axis=1)
    w2t = jnp.stack([w2t0, w2t1])

    xw = _k1_xw(x, w1p)
    src_r = edge_index[0].reshape(NBLKS, BLK)
    dst_r = edge_index[1].reshape(NBLKS, BLK)
    ea_r = edge_attr.reshape(NBLKS, BLK * NB)

    posp = jnp.pad(positions, ((0, 0), (0, 13)))
    partials = _sc_kernel(xw[0], xw[1], posp, batch,
                          src_r, dst_r, ea_r, w2t)
    return _k3_reduce(partials.reshape(NCORE, NSUB, G, 16))


# speedup vs baseline: 5.9369x; 1.2560x over previous
import functools

import jax
import jax.numpy as jnp
import numpy as np
from jax import lax
from jax.experimental import pallas as pl
from jax.experimental.pallas import tpu as pltpu
from jax.experimental.pallas import tpu_sc as plsc

N = 10000
NP = 10240
E = 160000
EP = 161792
G = 64
NB = 4
NA = 128

NCORE = 2
NSUB = 16
BLK = 32
NBLKS = EP // BLK
TPS = NBLKS // NSUB
NQ = TPS // 4
ROW = 112
XWC = 256
S3 = float(np.sqrt(3.0))
S5 = float(np.sqrt(5.0))


def _k1_xw(x, w1p):
    bn = 1280

    def body(x_ref, w_ref, o_ref):
        o_ref[...] = jnp.dot(
            x_ref[...], w_ref[0], preferred_element_type=jnp.float32
        )[None]

    return pl.pallas_call(
        body,
        grid=(NCORE, NP // bn),
        in_specs=[
            pl.BlockSpec((bn, NA), lambda c, j: (j, 0)),
            pl.BlockSpec((1, NA, XWC), lambda c, j: (c, 0, 0)),
        ],
        out_specs=pl.BlockSpec((1, bn, XWC), lambda c, j: (c, j, 0)),
        out_shape=jax.ShapeDtypeStruct((NCORE, NP, XWC), jnp.float32),
    )(x, w1p)


def _k3_reduce(partials):
    def body(p_ref, o_ref):
        acc = jnp.zeros((G, 16), jnp.float32)
        for c in range(NCORE):
            for s in range(NSUB):
                acc = acc + p_ref[c, s]
        o_ref[...] = jnp.sum(acc, axis=1, keepdims=True)

    return pl.pallas_call(
        body,
        out_shape=jax.ShapeDtypeStruct((G, 1), jnp.float32),
    )(partials)


def _sc_kernel(xwa, xwb, pos, batch, src_r, dst_r, ea, w2t):
    mesh = plsc.VectorSubcoreMesh(
        core_axis_name="c", subcore_axis_name="s", num_cores=NCORE,
        num_subcores=NSUB,
    )

    @functools.partial(
        pl.kernel,
        out_type=jax.ShapeDtypeStruct((NCORE, NSUB, G * 16), jnp.float32),
        mesh=mesh,
        compiler_params=pltpu.CompilerParams(
            needs_layout_passes=False, use_tc_tiling_on_sc=False),
        scratch_types=[
            pltpu.VMEM_SHARED((NP, ROW), jnp.float32),
            pltpu.VMEM((NP,), jnp.int32),
            pltpu.VMEM((NB, 64), jnp.float32),
            pltpu.VMEM((BLK,), jnp.int32),
            pltpu.VMEM((BLK,), jnp.int32),
            pltpu.VMEM((BLK,), jnp.int32),
            pltpu.VMEM((BLK,), jnp.int32),
            pltpu.VMEM((BLK,), jnp.int32),
            pltpu.VMEM((BLK,), jnp.int32),
            pltpu.VMEM((BLK,), jnp.int32),
            pltpu.VMEM((BLK,), jnp.int32),
            pltpu.VMEM((BLK * NB,), jnp.float32),
            pltpu.VMEM((BLK * NB,), jnp.float32),
            pltpu.VMEM((BLK, XWC), jnp.float32),
            pltpu.VMEM((BLK, XWC), jnp.float32),
            pltpu.VMEM((BLK, ROW), jnp.float32),
            pltpu.VMEM((BLK, ROW), jnp.float32),
            pltpu.VMEM((8 * BLK,), jnp.float32),
            pltpu.VMEM((G * 16,), jnp.float32),
            pltpu.VMEM((BLK, 16), jnp.float32),
            pltpu.VMEM((BLK, 16), jnp.float32),
            pltpu.VMEM((BLK, 16), jnp.float32),
            pltpu.VMEM((BLK, 16), jnp.float32),
            pltpu.SemaphoreType.DMA((18,)),
        ],
    )
    def body(xwa_h, xwb_h, pos_h, batch_h, src_h, dst_h, ea_h, w2t_h, out_h,
             nf_sh, batch_v, w2_v, sv0, sv1, sv2, sv3, dv0, dv1, dv2, dv3,
             eav0, eav1, xwv0, xwv1, mfv0, mfv1, sh_v, gacc_v,
             psv0, psv1, pdv0, pdv1, sem):
        cid = lax.axis_index("c")
        sid = lax.axis_index("s")

        SV = [sv0, sv1, sv2, sv3]
        DV = [dv0, dv1, dv2, dv3]
        EAV = [eav0, eav1]
        XWV = [xwv0, xwv1]
        MFV = [mfv0, mfv1]
        PSV = [psv0, psv1]
        PDV = [pdv0, pdv1]

        pltpu.sync_copy(batch_h, batch_v)
        pltpu.sync_copy(w2t_h.at[cid], w2_v)

        zero16 = jnp.zeros((16,), jnp.float32)
        ci32 = jnp.zeros((16,), jnp.int32)
        iota16 = lax.iota(jnp.int32, 16)

        def zrow(i, _):
            for o in range(0, ROW, 16):
                mfv0[i, pl.ds(o, 16)] = zero16
            return 0

        lax.fori_loop(0, BLK, zrow, 0)

        for o in range(0, G * 16, 16):
            gacc_v[pl.ds(o, 16)] = zero16

        rows_per = NP // NSUB
        for ch in range(rows_per // BLK):
            pltpu.sync_copy(
                mfv0.at[pl.ds(0, BLK)],
                nf_sh.at[pl.ds(sid * rows_per + ch * BLK, BLK)],
            )
        plsc.subcore_barrier()

        def bcast(ref, idx):
            return plsc.load_gather(ref, [ci32 + idx])

        def idx_start(kq, c):
            b = sid + kq * NSUB
            pltpu.async_copy(src_h.at[b], SV[c], sem.at[c])
            pltpu.async_copy(dst_h.at[b], DV[c], sem.at[4 + c])

        def idx_wait(kq, c):
            b = sid + kq * NSUB
            pltpu.make_async_copy(src_h.at[b], SV[c], sem.at[c]).wait()
            pltpu.make_async_copy(dst_h.at[b], DV[c], sem.at[4 + c]).wait()

        def rows_start(kq, c, stage_a):
            p = c & 1
            b = sid + kq * NSUB
            pltpu.async_copy(ea_h.at[b], EAV[p], sem.at[8 + p])
            if stage_a:
                @pl.when(cid == 0)
                def _():
                    pltpu.async_copy(xwa_h.at[SV[c]], XWV[p], sem.at[10 + p])

                @pl.when(cid == 1)
                def _():
                    pltpu.async_copy(xwb_h.at[SV[c]], XWV[p], sem.at[10 + p])
            else:
                pltpu.async_copy(nf_sh.at[SV[c]], MFV[p], sem.at[10 + p])
            pltpu.async_copy(pos_h.at[SV[c]], PSV[p], sem.at[12 + p])
            pltpu.async_copy(pos_h.at[DV[c]], PDV[p], sem.at[14 + p])

        def rows_wait(kq, c, stage_a):
            p = c & 1
            b = sid + kq * NSUB
            pltpu.make_async_copy(ea_h.at[b], EAV[p], sem.at[8 + p]).wait()
            if stage_a:
                @pl.when(cid == 0)
                def _():
                    pltpu.make_async_copy(
                        xwa_h.at[SV[c]], XWV[p], sem.at[10 + p]).wait()

                @pl.when(cid == 1)
                def _():
                    pltpu.make_async_copy(
                        xwb_h.at[SV[c]], XWV[p], sem.at[10 + p]).wait()
            else:
                pltpu.make_async_copy(
                    nf_sh.at[SV[c]], MFV[p], sem.at[10 + p]).wait()
            pltpu.make_async_copy(
                pos_h.at[SV[c]], PSV[p], sem.at[12 + p]).wait()
            pltpu.make_async_copy(
                pos_h.at[DV[c]], PDV[p], sem.at[14 + p]).wait()

        def scatter_start(c):
            p = c & 1
            pltpu.async_copy(MFV[p], nf_sh.at[DV[c]], sem.at[16 + p],
                             add=True)

        def scatter_wait(c):
            p = c & 1
            pltpu.make_async_copy(MFV[p], nf_sh.at[DV[c]],
                                  sem.at[16 + p]).wait()

        def sh_pass(p, want1, want2):
            for g in range(BLK // 16):
                r16 = iota16 + g * 16
                exv = (plsc.load_gather(PSV[p], [r16, ci32])
                       - plsc.load_gather(PDV[p], [r16, ci32]))
                eyv = (plsc.load_gather(PSV[p], [r16, ci32 + 1])
                       - plsc.load_gather(PDV[p], [r16, ci32 + 1]))
                ezv = (plsc.load_gather(PSV[p], [r16, ci32 + 2])
                       - plsc.load_gather(PDV[p], [r16, ci32 + 2]))
                if want1:
                    sh_v[pl.ds(0 * BLK + g * 16, 16)] = S3 * exv
                    sh_v[pl.ds(1 * BLK + g * 16, 16)] = S3 * eyv
                    sh_v[pl.ds(2 * BLK + g * 16, 16)] = S3 * ezv
                if want2:
                    sh_v[pl.ds(3 * BLK + g * 16, 16)] = (S5 * S3) * (exv * ezv)
                    sh_v[pl.ds(4 * BLK + g * 16, 16)] = (S5 * S3) * (exv * eyv)
                    sh_v[pl.ds(5 * BLK + g * 16, 16)] = S5 * (
                        eyv * eyv - 0.5 * (exv * exv + ezv * ezv)
                    )
                    sh_v[pl.ds(6 * BLK + g * 16, 16)] = (S5 * S3) * (eyv * ezv)
                    sh_v[pl.ds(7 * BLK + g * 16, 16)] = (S5 * 0.5 * S3) * (
                        ezv * ezv - exv * exv
                    )

        def ea4(p, i):
            return [bcast(EAV[p], i * NB + v) for v in range(NB)]

        def q4(p, i):
            e = ea4(p, i)
            q = []
            for j in range(4):
                acc = XWV[p][i, pl.ds(0 * 64 + j * 16, 16)] * e[0]
                acc += XWV[p][i, pl.ds(1 * 64 + j * 16, 16)] * e[1]
                acc += XWV[p][i, pl.ds(2 * 64 + j * 16, 16)] * e[2]
                acc += XWV[p][i, pl.ds(3 * 64 + j * 16, 16)] * e[3]
                q.append(acc)
            return q

        def edge_a0(p):
            def edge(i, _):
                q = q4(p, i)
                MFV[p][i, pl.ds(0, 16)] = q[0]
                MFV[p][i, pl.ds(16, 16)] = q[1]
                for m in range(3):
                    sm = bcast(sh_v, m * BLK + i)
                    MFV[p][i, pl.ds(32 + 24 * m, 16)] = q[2] * sm
                    MFV[p][i, pl.ds(48 + 24 * m, 16)] = q[3] * sm
                return 0

            lax.fori_loop(0, BLK, edge, 0)

        def edge_a1(p):
            def edge(i, _):
                q = q4(p, i)
                MFV[p][i, pl.ds(0, 16)] = q[0]
                MFV[p][i, pl.ds(16, 16)] = q[1]
                for m in range(5):
                    sm = bcast(sh_v, (3 + m) * BLK + i)
                    MFV[p][i, pl.ds(32 + 16 * m, 16)] = q[2] * sm
                return 0

            lax.fori_loop(0, BLK, edge, 0)

        def a4(p, i):
            e = ea4(p, i)
            a = []
            for j in range(4):
                acc = w2_v[0, pl.ds(j * 16, 16)] * e[0]
                acc += w2_v[1, pl.ds(j * 16, 16)] * e[1]
                acc += w2_v[2, pl.ds(j * 16, 16)] * e[2]
                acc += w2_v[3, pl.ds(j * 16, 16)] * e[3]
                a.append(acc)
            return a

        def gupd(c, i, acc):
            d16 = bcast(DV[c], i)
            g16 = plsc.load_gather(batch_v, [d16])
            plsc.addupdate_scatter(gacc_v, [g16 * 16 + iota16], acc)

        def edge_b0(p, c):
            def edge(i, _):
                a = a4(p, i)
                acc = MFV[p][i, pl.ds(0, 16)] * a[0]
                acc += MFV[p][i, pl.ds(16, 16)] * a[1]
                for m in range(3):
                    sm = bcast(sh_v, m * BLK + i)
                    acc += (MFV[p][i, pl.ds(32 + 24 * m, 16)] * a[2]) * sm
                    acc += (MFV[p][i, pl.ds(48 + 24 * m, 16)] * a[3]) * sm
                gupd(c, i, acc)
                return 0

            lax.fori_loop(0, BLK, edge, 0)

        def edge_b1(p, c):
            def edge(i, _):
                a = a4(p, i)
                acc = MFV[p][i, pl.ds(0, 16)] * a[0]
                acc += MFV[p][i, pl.ds(16, 16)] * a[1]
                for m in range(5):
                    sm = bcast(sh_v, (3 + m) * BLK + i)
                    acc += (MFV[p][i, pl.ds(32 + 16 * m, 16)] * a[2]) * sm
                gupd(c, i, acc)
                return 0

            lax.fori_loop(0, BLK, edge, 0)

        def block_a(kq, c, first, n1, n2):
            p = c & 1
            if n2:
                idx_start(kq + 2, (c + 2) % 4)
            rows_wait(kq, c, True)
            if n1:
                idx_wait(kq + 1, (c + 1) % 4)
                rows_start(kq + 1, (c + 1) % 4, True)

            @pl.when(cid == 0)
            def _():
                sh_pass(p, True, False)
                edge_a0(p)

            @pl.when(cid == 1)
            def _():
                sh_pass(p, False, True)
                edge_a1(p)

            if not first:
                scatter_wait((c - 1) % 4)
            scatter_start(c)

        def block_b(kq, c, n1, n2):
            p = c & 1
            if n2:
                idx_start(kq + 2, (c + 2) % 4)
            rows_wait(kq, c, False)
            if n1:
                idx_wait(kq + 1, (c + 1) % 4)
                rows_start(kq + 1, (c + 1) % 4, False)

            @pl.when(cid == 0)
            def _():
                sh_pass(p, True, False)
                edge_b0(p, c)

            @pl.when(cid == 1)
            def _():
                sh_pass(p, False, True)
                edge_b1(p, c)

        idx_start(0, 0)
        idx_start(1, 1)
        idx_wait(0, 0)
        rows_start(0, 0, True)
        block_a(0, 0, True, True, True)
        block_a(1, 1, False, True, True)
        block_a(2, 2, False, True, True)
        block_a(3, 3, False, True, True)

        def quad_a(q, _):
            k0 = q * 4
            block_a(k0, 0, False, True, True)
            block_a(k0 + 1, 1, False, True, True)
            block_a(k0 + 2, 2, False, True, True)
            block_a(k0 + 3, 3, False, True, True)
            return 0

        lax.fori_loop(1, NQ - 1, quad_a, 0)

        kl = (NQ - 1) * 4
        block_a(kl, 0, False, True, True)
        block_a(kl + 1, 1, False, True, True)
        block_a(kl + 2, 2, False, True, False)
        block_a(kl + 3, 3, False, False, False)
        scatter_wait(3)
        plsc.subcore_barrier()

        idx_start(0, 0)
        idx_start(1, 1)
        idx_wait(0, 0)
        rows_start(0, 0, False)
        block_b(0, 0, True, True)
        block_b(1, 1, True, True)
        block_b(2, 2, True, True)
        block_b(3, 3, True, True)

        def quad_b(q, _):
            k0 = q * 4
            block_b(k0, 0, True, True)
            block_b(k0 + 1, 1, True, True)
            block_b(k0 + 2, 2, True, True)
            block_b(k0 + 3, 3, True, True)
            return 0

        lax.fori_loop(1, NQ - 1, quad_b, 0)

        kl = (NQ - 1) * 4
        block_b(kl, 0, True, True)
        block_b(kl + 1, 1, True, True)
        block_b(kl + 2, 2, True, False)
        block_b(kl + 3, 3, False, False)

        pltpu.sync_copy(gacc_v, out_h.at[cid, sid])

    return body(xwa, xwb, pos, batch, src_r, dst_r, ea, w2t)


def kernel(positions, x, edge_attr, edge_index, batch, W1_0, W1_1, W1_2,
           W2_0, W2_1, W2_2):
    inv1 = 1.0 / float(np.sqrt(NA * NB))
    fan2 = 416.0
    k0 = 1.0 / float(np.sqrt(fan2))
    k1 = 1.0 / float(np.sqrt(fan2 * 3.0))
    k2 = 1.0 / float(np.sqrt(fan2 * 5.0))

    w1cat = jnp.concatenate([W1_0, W1_1, W1_2], axis=2) * inv1
    zpad8 = jnp.zeros((NA, 8), jnp.float32)
    zpad16 = jnp.zeros((NA, 16), jnp.float32)
    segs0 = [jnp.concatenate(
        [w1cat[:, v, 0:32], w1cat[:, v, 64:88], zpad8], axis=1)
        for v in range(NB)]
    segs1 = [jnp.concatenate(
        [w1cat[:, v, 32:64], w1cat[:, v, 88:104], zpad16], axis=1)
        for v in range(NB)]
    w1p0 = jnp.concatenate(segs0, axis=1)
    w1p1 = jnp.concatenate(segs1, axis=1)
    w1p = jnp.stack([w1p0, w1p1])

    z8 = jnp.zeros((NB, 8), jnp.float32)
    z16 = jnp.zeros((NB, 16), jnp.float32)
    w2t0 = jnp.concatenate(
        [W2_0[0:32, :, 0].T * k0, W2_1[:, :, 0].T * k1, z8], axis=1)
    w2t1 = jnp.concatenate(
        [W2_0[32:64, :, 0].T * k0, W2_2[:, :, 0].T * k2, z16], axis=1)
    w2t = jnp.stack([w2t0, w2t1])

    xp = jnp.concatenate(
        [x, jnp.zeros((NP - N, NA), jnp.float32)], axis=0)
    xw = _k1_xw(xp, w1p)

    pad_i = jnp.full((EP - E,), N, edge_index.dtype)
    src_r = jnp.concatenate([edge_index[0], pad_i]).reshape(NBLKS, BLK)
    dst_r = jnp.concatenate([edge_index[1], pad_i]).reshape(NBLKS, BLK)
    ea_r = jnp.concatenate(
        [edge_attr, jnp.zeros((EP - E, NB), edge_attr.dtype)]
    ).reshape(NBLKS, BLK * NB)
    batch_p = jnp.concatenate(
        [batch, jnp.zeros((NP - N,), batch.dtype)])
    posp = jnp.pad(positions, ((0, NP - N), (0, 13)))

    partials = _sc_kernel(xw[0], xw[1], posp, batch_p,
                          src_r, dst_r, ea_r, w2t)
    return _k3_reduce(partials.reshape(NCORE, NSUB, G, 16))


# speedup vs baseline: 5.9379x; 1.0002x over previous
import functools

import jax
import jax.numpy as jnp
import numpy as np
from jax import lax
from jax.experimental import pallas as pl
from jax.experimental.pallas import tpu as pltpu
from jax.experimental.pallas import tpu_sc as plsc

N = 10000
NP = 10240
E = 160000
EP = 161792
G = 64
NB = 4
NA = 128

NCORE = 2
NSUB = 16
BLK = 32
NBLKS = EP // BLK
TPS = NBLKS // NSUB
NQ = TPS // 4
ROW = 112
XWC = 256
S3 = float(np.sqrt(3.0))
S5 = float(np.sqrt(5.0))


def _k1_xw(x, w1p):
    bn = 1280

    def body(x_ref, w_ref, o_ref):
        o_ref[...] = jnp.dot(
            x_ref[...], w_ref[0], preferred_element_type=jnp.float32
        )[None]

    return pl.pallas_call(
        body,
        grid=(NCORE, NP // bn),
        in_specs=[
            pl.BlockSpec((bn, NA), lambda c, j: (j, 0)),
            pl.BlockSpec((1, NA, XWC), lambda c, j: (c, 0, 0)),
        ],
        out_specs=pl.BlockSpec((1, bn, XWC), lambda c, j: (c, j, 0)),
        out_shape=jax.ShapeDtypeStruct((NCORE, NP, XWC), jnp.float32),
    )(x, w1p)


def _k3_reduce(partials):
    def body(p_ref, o_ref):
        acc = jnp.zeros((G, 16), jnp.float32)
        for c in range(NCORE):
            for s in range(NSUB):
                acc = acc + p_ref[c, s]
        o_ref[...] = jnp.sum(acc, axis=1, keepdims=True)

    return pl.pallas_call(
        body,
        out_shape=jax.ShapeDtypeStruct((G, 1), jnp.float32),
    )(partials)


def _sc_kernel(xwa, xwb, pos, batch, src_r, dst_r, ea, w2t):
    mesh = plsc.VectorSubcoreMesh(
        core_axis_name="c", subcore_axis_name="s", num_cores=NCORE,
        num_subcores=NSUB,
    )

    @functools.partial(
        pl.kernel,
        out_type=jax.ShapeDtypeStruct((NCORE, NSUB, G * 16), jnp.float32),
        mesh=mesh,
        compiler_params=pltpu.CompilerParams(
            needs_layout_passes=False, use_tc_tiling_on_sc=False),
        scratch_types=[
            pltpu.VMEM_SHARED((NP, ROW), jnp.float32),
            pltpu.VMEM((NP,), jnp.int32),
            pltpu.VMEM((NB, 64), jnp.float32),
            pltpu.VMEM((BLK,), jnp.int32),
            pltpu.VMEM((BLK,), jnp.int32),
            pltpu.VMEM((BLK,), jnp.int32),
            pltpu.VMEM((BLK,), jnp.int32),
            pltpu.VMEM((BLK,), jnp.int32),
            pltpu.VMEM((BLK,), jnp.int32),
            pltpu.VMEM((BLK,), jnp.int32),
            pltpu.VMEM((BLK,), jnp.int32),
            pltpu.VMEM((BLK * NB,), jnp.float32),
            pltpu.VMEM((BLK * NB,), jnp.float32),
            pltpu.VMEM((BLK, XWC), jnp.float32),
            pltpu.VMEM((BLK, XWC), jnp.float32),
            pltpu.VMEM((BLK, ROW), jnp.float32),
            pltpu.VMEM((BLK, ROW), jnp.float32),
            pltpu.VMEM((8 * BLK,), jnp.float32),
            pltpu.VMEM((G * 16,), jnp.float32),
            pltpu.VMEM((BLK, 16), jnp.float32),
            pltpu.VMEM((BLK, 16), jnp.float32),
            pltpu.VMEM((BLK, 16), jnp.float32),
            pltpu.VMEM((BLK, 16), jnp.float32),
            pltpu.SemaphoreType.DMA((18,)),
        ],
    )
    def body(xwa_h, xwb_h, pos_h, batch_h, src_h, dst_h, ea_h, w2t_h, out_h,
             nf_sh, batch_v, w2_v, sv0, sv1, sv2, sv3, dv0, dv1, dv2, dv3,
             eav0, eav1, xwv0, xwv1, mfv0, mfv1, sh_v, gacc_v,
             psv0, psv1, pdv0, pdv1, sem):
        cid = lax.axis_index("c")
        sid = lax.axis_index("s")

        SV = [sv0, sv1, sv2, sv3]
        DV = [dv0, dv1, dv2, dv3]
        EAV = [eav0, eav1]
        XWV = [xwv0, xwv1]
        MFV = [mfv0, mfv1]
        PSV = [psv0, psv1]
        PDV = [pdv0, pdv1]

        pltpu.sync_copy(batch_h, batch_v)
        pltpu.sync_copy(w2t_h.at[cid], w2_v)

        zero16 = jnp.zeros((16,), jnp.float32)
        ci32 = jnp.zeros((16,), jnp.int32)
        iota16 = lax.iota(jnp.int32, 16)

        def zrow(i, _):
            for o in range(0, ROW, 16):
                mfv0[i, pl.ds(o, 16)] = zero16
            return 0

        lax.fori_loop(0, BLK, zrow, 0)

        for o in range(0, G * 16, 16):
            gacc_v[pl.ds(o, 16)] = zero16

        rows_per = NP // NSUB
        for ch in range(rows_per // BLK):
            pltpu.sync_copy(
                mfv0.at[pl.ds(0, BLK)],
                nf_sh.at[pl.ds(sid * rows_per + ch * BLK, BLK)],
            )
        plsc.subcore_barrier()

        def bcast(ref, idx):
            return plsc.load_gather(ref, [ci32 + idx])

        def idx_start(kq, c):
            b = sid + kq * NSUB
            pltpu.async_copy(src_h.at[b], SV[c], sem.at[c])
            pltpu.async_copy(dst_h.at[b], DV[c], sem.at[4 + c])

        def idx_wait(kq, c):
            b = sid + kq * NSUB
            pltpu.make_async_copy(src_h.at[b], SV[c], sem.at[c]).wait()
            pltpu.make_async_copy(dst_h.at[b], DV[c], sem.at[4 + c]).wait()

        def rows_start(kq, c, stage_a):
            p = c & 1
            b = sid + kq * NSUB
            pltpu.async_copy(ea_h.at[b], EAV[p], sem.at[8 + p])
            if stage_a:
                @pl.when(cid == 0)
                def _():
                    pltpu.async_copy(xwa_h.at[SV[c]], XWV[p], sem.at[10 + p])

                @pl.when(cid == 1)
                def _():
                    pltpu.async_copy(xwb_h.at[SV[c]], XWV[p], sem.at[10 + p])
            else:
                pltpu.async_copy(nf_sh.at[SV[c]], MFV[p], sem.at[10 + p])
            pltpu.async_copy(pos_h.at[SV[c]], PSV[p], sem.at[12 + p])
            pltpu.async_copy(pos_h.at[DV[c]], PDV[p], sem.at[14 + p])

        def rows_wait(kq, c, stage_a):
            p = c & 1
            b = sid + kq * NSUB
            pltpu.make_async_copy(ea_h.at[b], EAV[p], sem.at[8 + p]).wait()
            if stage_a:
                @pl.when(cid == 0)
                def _():
                    pltpu.make_async_copy(
                        xwa_h.at[SV[c]], XWV[p], sem.at[10 + p]).wait()

                @pl.when(cid == 1)
                def _():
                    pltpu.make_async_copy(
                        xwb_h.at[SV[c]], XWV[p], sem.at[10 + p]).wait()
            else:
                pltpu.make_async_copy(
                    nf_sh.at[SV[c]], MFV[p], sem.at[10 + p]).wait()
            pltpu.make_async_copy(
                pos_h.at[SV[c]], PSV[p], sem.at[12 + p]).wait()
            pltpu.make_async_copy(
                pos_h.at[DV[c]], PDV[p], sem.at[14 + p]).wait()

        def scatter_start(c):
            p = c & 1
            pltpu.async_copy(MFV[p], nf_sh.at[DV[c]], sem.at[16 + p],
                             add=True)

        def scatter_wait(c):
            p = c & 1
            pltpu.make_async_copy(MFV[p], nf_sh.at[DV[c]],
                                  sem.at[16 + p]).wait()

        def sh_pass(p, want1, want2):
            for g in range(BLK // 16):
                r16 = iota16 + g * 16
                exv = (plsc.load_gather(PSV[p], [r16, ci32])
                       - plsc.load_gather(PDV[p], [r16, ci32]))
                eyv = (plsc.load_gather(PSV[p], [r16, ci32 + 1])
                       - plsc.load_gather(PDV[p], [r16, ci32 + 1]))
                ezv = (plsc.load_gather(PSV[p], [r16, ci32 + 2])
                       - plsc.load_gather(PDV[p], [r16, ci32 + 2]))
                if want1:
                    sh_v[pl.ds(0 * BLK + g * 16, 16)] = S3 * exv
                    sh_v[pl.ds(1 * BLK + g * 16, 16)] = S3 * eyv
                    sh_v[pl.ds(2 * BLK + g * 16, 16)] = S3 * ezv
                if want2:
                    sh_v[pl.ds(3 * BLK + g * 16, 16)] = (S5 * S3) * (exv * ezv)
                    sh_v[pl.ds(4 * BLK + g * 16, 16)] = (S5 * S3) * (exv * eyv)
                    sh_v[pl.ds(5 * BLK + g * 16, 16)] = S5 * (
                        eyv * eyv - 0.5 * (exv * exv + ezv * ezv)
                    )
                    sh_v[pl.ds(6 * BLK + g * 16, 16)] = (S5 * S3) * (eyv * ezv)
                    sh_v[pl.ds(7 * BLK + g * 16, 16)] = (S5 * 0.5 * S3) * (
                        ezv * ezv - exv * exv
                    )

        def ea4(p, i):
            return [bcast(EAV[p], i * NB + v) for v in range(NB)]

        def q4(p, i, nj):
            e = ea4(p, i)
            q = []
            for j in range(nj):
                acc = XWV[p][i, pl.ds(0 * 64 + j * 16, 16)] * e[0]
                acc += XWV[p][i, pl.ds(1 * 64 + j * 16, 16)] * e[1]
                acc += XWV[p][i, pl.ds(2 * 64 + j * 16, 16)] * e[2]
                acc += XWV[p][i, pl.ds(3 * 64 + j * 16, 16)] * e[3]
                q.append(acc)
            return q

        def edge_a0(p):
            def edge(i, _):
                q = q4(p, i, 4)
                MFV[p][i, pl.ds(0, 16)] = q[0]
                MFV[p][i, pl.ds(16, 16)] = q[1]
                for m in range(3):
                    sm = bcast(sh_v, m * BLK + i)
                    MFV[p][i, pl.ds(32 + 24 * m, 16)] = q[2] * sm
                    MFV[p][i, pl.ds(48 + 24 * m, 16)] = q[3] * sm
                return 0

            lax.fori_loop(0, BLK, edge, 0)

        def edge_a1(p):
            def edge(i, _):
                q = q4(p, i, 3)
                MFV[p][i, pl.ds(0, 16)] = q[0]
                MFV[p][i, pl.ds(16, 16)] = q[1]
                for m in range(5):
                    sm = bcast(sh_v, (3 + m) * BLK + i)
                    MFV[p][i, pl.ds(32 + 16 * m, 16)] = q[2] * sm
                return 0

            lax.fori_loop(0, BLK, edge, 0)

        def a4(p, i, nj):
            e = ea4(p, i)
            a = []
            for j in range(nj):
                acc = w2_v[0, pl.ds(j * 16, 16)] * e[0]
                acc += w2_v[1, pl.ds(j * 16, 16)] * e[1]
                acc += w2_v[2, pl.ds(j * 16, 16)] * e[2]
                acc += w2_v[3, pl.ds(j * 16, 16)] * e[3]
                a.append(acc)
            return a

        def gupd(c, i, acc):
            d16 = bcast(DV[c], i)
            g16 = plsc.load_gather(batch_v, [d16])
            plsc.addupdate_scatter(gacc_v, [g16 * 16 + iota16], acc)

        def edge_b0(p, c):
            def edge(i, _):
                a = a4(p, i, 4)
                acc = MFV[p][i, pl.ds(0, 16)] * a[0]
                acc += MFV[p][i, pl.ds(16, 16)] * a[1]
                for m in range(3):
                    sm = bcast(sh_v, m * BLK + i)
                    acc += (MFV[p][i, pl.ds(32 + 24 * m, 16)] * a[2]) * sm
                    acc += (MFV[p][i, pl.ds(48 + 24 * m, 16)] * a[3]) * sm
                gupd(c, i, acc)
                return 0

            lax.fori_loop(0, BLK, edge, 0)

        def edge_b1(p, c):
            def edge(i, _):
                a = a4(p, i, 3)
                acc = MFV[p][i, pl.ds(0, 16)] * a[0]
                acc += MFV[p][i, pl.ds(16, 16)] * a[1]
                for m in range(5):
                    sm = bcast(sh_v, (3 + m) * BLK + i)
                    acc += (MFV[p][i, pl.ds(32 + 16 * m, 16)] * a[2]) * sm
                gupd(c, i, acc)
                return 0

            lax.fori_loop(0, BLK, edge, 0)

        def block_a(kq, c, first, n1, n2):
            p = c & 1
            if n2:
                idx_start(kq + 2, (c + 2) % 4)
            rows_wait(kq, c, True)
            if n1:
                idx_wait(kq + 1, (c + 1) % 4)
                rows_start(kq + 1, (c + 1) % 4, True)

            @pl.when(cid == 0)
            def _():
                sh_pass(p, True, False)
                edge_a0(p)

            @pl.when(cid == 1)
            def _():
                sh_pass(p, False, True)
                edge_a1(p)

            if not first:
                scatter_wait((c - 1) % 4)
            scatter_start(c)

        def block_b(kq, c, n1, n2):
            p = c & 1
            if n2:
                idx_start(kq + 2, (c + 2) % 4)
            rows_wait(kq, c, False)
            if n1:
                idx_wait(kq + 1, (c + 1) % 4)
                rows_start(kq + 1, (c + 1) % 4, False)

            @pl.when(cid == 0)
            def _():
                sh_pass(p, True, False)
                edge_b0(p, c)

            @pl.when(cid == 1)
            def _():
                sh_pass(p, False, True)
                edge_b1(p, c)

        idx_start(0, 0)
        idx_start(1, 1)
        idx_wait(0, 0)
        rows_start(0, 0, True)
        block_a(0, 0, True, True, True)
        block_a(1, 1, False, True, True)
        block_a(2, 2, False, True, True)
        block_a(3, 3, False, True, True)

        def quad_a(q, _):
            k0 = q * 4
            block_a(k0, 0, False, True, True)
            block_a(k0 + 1, 1, False, True, True)
            block_a(k0 + 2, 2, False, True, True)
            block_a(k0 + 3, 3, False, True, True)
            return 0

        lax.fori_loop(1, NQ - 1, quad_a, 0)

        kl = (NQ - 1) * 4
        block_a(kl, 0, False, True, True)
        block_a(kl + 1, 1, False, True, True)
        block_a(kl + 2, 2, False, True, False)
        block_a(kl + 3, 3, False, False, False)
        scatter_wait(3)
        plsc.subcore_barrier()

        idx_start(0, 0)
        idx_start(1, 1)
        idx_wait(0, 0)
        rows_start(0, 0, False)
        block_b(0, 0, True, True)
        block_b(1, 1, True, True)
        block_b(2, 2, True, True)
        block_b(3, 3, True, True)

        def quad_b(q, _):
            k0 = q * 4
            block_b(k0, 0, True, True)
            block_b(k0 + 1, 1, True, True)
            block_b(k0 + 2, 2, True, True)
            block_b(k0 + 3, 3, True, True)
            return 0

        lax.fori_loop(1, NQ - 1, quad_b, 0)

        kl = (NQ - 1) * 4
        block_b(kl, 0, True, True)
        block_b(kl + 1, 1, True, True)
        block_b(kl + 2, 2, True, False)
        block_b(kl + 3, 3, False, False)

        pltpu.sync_copy(gacc_v, out_h.at[cid, sid])

    return body(xwa, xwb, pos, batch, src_r, dst_r, ea, w2t)


def kernel(positions, x, edge_attr, edge_index, batch, W1_0, W1_1, W1_2,
           W2_0, W2_1, W2_2):
    inv1 = 1.0 / float(np.sqrt(NA * NB))
    fan2 = 416.0
    k0 = 1.0 / float(np.sqrt(fan2))
    k1 = 1.0 / float(np.sqrt(fan2 * 3.0))
    k2 = 1.0 / float(np.sqrt(fan2 * 5.0))

    w1cat = jnp.concatenate([W1_0, W1_1, W1_2], axis=2) * inv1
    zpad8 = jnp.zeros((NA, 8), jnp.float32)
    zpad16 = jnp.zeros((NA, 16), jnp.float32)
    segs0 = [jnp.concatenate(
        [w1cat[:, v, 0:32], w1cat[:, v, 64:88], zpad8], axis=1)
        for v in range(NB)]
    segs1 = [jnp.concatenate(
        [w1cat[:, v, 32:64], w1cat[:, v, 88:104], zpad16], axis=1)
        for v in range(NB)]
    w1p0 = jnp.concatenate(segs0, axis=1)
    w1p1 = jnp.concatenate(segs1, axis=1)
    w1p = jnp.stack([w1p0, w1p1])

    z8 = jnp.zeros((NB, 8), jnp.float32)
    z16 = jnp.zeros((NB, 16), jnp.float32)
    w2t0 = jnp.concatenate(
        [W2_0[0:32, :, 0].T * k0, W2_1[:, :, 0].T * k1, z8], axis=1)
    w2t1 = jnp.concatenate(
        [W2_0[32:64, :, 0].T * k0, W2_2[:, :, 0].T * k2, z16], axis=1)
    w2t = jnp.stack([w2t0, w2t1])

    xp = jnp.concatenate(
        [x, jnp.zeros((NP - N, NA), jnp.float32)], axis=0)
    xw = _k1_xw(xp, w1p)

    pad_i = jnp.full((EP - E,), N, edge_index.dtype)
    src_r = jnp.concatenate([edge_index[0], pad_i]).reshape(NBLKS, BLK)
    dst_r = jnp.concatenate([edge_index[1], pad_i]).reshape(NBLKS, BLK)
    ea_r = jnp.concatenate(
        [edge_attr, jnp.zeros((EP - E, NB), edge_attr.dtype)]
    ).reshape(NBLKS, BLK * NB)
    batch_p = jnp.concatenate(
        [batch, jnp.zeros((NP - N,), batch.dtype)])
    posp = jnp.pad(positions, ((0, NP - N), (0, 13)))

    partials = _sc_kernel(xw[0], xw[1], posp, batch_p,
                          src_r, dst_r, ea_r, w2t)
    return _k3_reduce(partials.reshape(NCORE, NSUB, G, 16))


# speedup vs baseline: 5.9419x; 1.0007x over previous
import functools

import jax
import jax.numpy as jnp
import numpy as np
from jax import lax
from jax.experimental import pallas as pl
from jax.experimental.pallas import tpu as pltpu
from jax.experimental.pallas import tpu_sc as plsc

N = 10000
NP = 10240
E = 160000
EP = 161792
G = 64
NB = 4
NA = 128

NCORE = 2
NSUB = 16
BLK = 32
NBLKS = EP // BLK
TPS = NBLKS // NSUB
NQ = TPS // 4
ROW = 112
XWC = 256
S3 = float(np.sqrt(3.0))
S5 = float(np.sqrt(5.0))


def _k1_xw(x, w1p):
    bn = 1280

    def body(x_ref, w_ref, o_ref):
        o_ref[...] = jnp.dot(
            x_ref[...], w_ref[0], preferred_element_type=jnp.float32
        )[None]

    return pl.pallas_call(
        body,
        grid=(NCORE, NP // bn),
        in_specs=[
            pl.BlockSpec((bn, NA), lambda c, j: (j, 0)),
            pl.BlockSpec((1, NA, XWC), lambda c, j: (c, 0, 0)),
        ],
        out_specs=pl.BlockSpec((1, bn, XWC), lambda c, j: (c, j, 0)),
        out_shape=jax.ShapeDtypeStruct((NCORE, NP, XWC), jnp.float32),
    )(x, w1p)


def _k3_reduce(partials):
    def body(p_ref, o_ref):
        acc = jnp.zeros((G, 16), jnp.float32)
        for c in range(NCORE):
            for s in range(NSUB):
                acc = acc + p_ref[c, s]
        o_ref[...] = jnp.sum(acc, axis=1, keepdims=True)

    return pl.pallas_call(
        body,
        out_shape=jax.ShapeDtypeStruct((G, 1), jnp.float32),
    )(partials)


def _sc_kernel(xwa, xwb, pos, batch, sd_r, ea, w2t):
    mesh = plsc.VectorSubcoreMesh(
        core_axis_name="c", subcore_axis_name="s", num_cores=NCORE,
        num_subcores=NSUB,
    )

    @functools.partial(
        pl.kernel,
        out_type=jax.ShapeDtypeStruct((NCORE, NSUB, G * 16), jnp.float32),
        mesh=mesh,
        compiler_params=pltpu.CompilerParams(
            needs_layout_passes=False, use_tc_tiling_on_sc=False),
        scratch_types=[
            pltpu.VMEM_SHARED((NP, ROW), jnp.float32),
            pltpu.VMEM((NP,), jnp.int32),
            pltpu.VMEM((NB, 64), jnp.float32),
            pltpu.VMEM((2 * BLK,), jnp.int32),
            pltpu.VMEM((2 * BLK,), jnp.int32),
            pltpu.VMEM((2 * BLK,), jnp.int32),
            pltpu.VMEM((2 * BLK,), jnp.int32),
            pltpu.VMEM((BLK * NB,), jnp.float32),
            pltpu.VMEM((BLK * NB,), jnp.float32),
            pltpu.VMEM((BLK, XWC), jnp.float32),
            pltpu.VMEM((BLK, XWC), jnp.float32),
            pltpu.VMEM((BLK, ROW), jnp.float32),
            pltpu.VMEM((BLK, ROW), jnp.float32),
            pltpu.VMEM((8 * BLK,), jnp.float32),
            pltpu.VMEM((G * 16,), jnp.float32),
            pltpu.VMEM((2 * BLK, 16), jnp.float32),
            pltpu.VMEM((2 * BLK, 16), jnp.float32),
            pltpu.SemaphoreType.DMA((18,)),
        ],
    )
    def body(xwa_h, xwb_h, pos_h, batch_h, sd_h, ea_h, w2t_h, out_h,
             nf_sh, batch_v, w2_v, sd0, sd1, sd2, sd3,
             eav0, eav1, xwv0, xwv1, mfv0, mfv1, sh_v, gacc_v,
             psd0, psd1, sem):
        cid = lax.axis_index("c")
        sid = lax.axis_index("s")

        SDV = [sd0, sd1, sd2, sd3]
        EAV = [eav0, eav1]
        XWV = [xwv0, xwv1]
        MFV = [mfv0, mfv1]
        PSD = [psd0, psd1]

        pltpu.sync_copy(batch_h, batch_v)
        pltpu.sync_copy(w2t_h.at[cid], w2_v)

        zero16 = jnp.zeros((16,), jnp.float32)
        ci32 = jnp.zeros((16,), jnp.int32)
        iota16 = lax.iota(jnp.int32, 16)

        def zrow(i, _):
            for o in range(0, ROW, 16):
                mfv0[i, pl.ds(o, 16)] = zero16
            return 0

        lax.fori_loop(0, BLK, zrow, 0)

        for o in range(0, G * 16, 16):
            gacc_v[pl.ds(o, 16)] = zero16

        rows_per = NP // NSUB
        for ch in range(rows_per // BLK):
            pltpu.sync_copy(
                mfv0.at[pl.ds(0, BLK)],
                nf_sh.at[pl.ds(sid * rows_per + ch * BLK, BLK)],
            )
        plsc.subcore_barrier()

        def bcast(ref, idx):
            return plsc.load_gather(ref, [ci32 + idx])

        def idx_start(kq, c):
            b = sid + kq * NSUB
            pltpu.async_copy(sd_h.at[b], SDV[c], sem.at[c])

        def idx_wait(kq, c):
            b = sid + kq * NSUB
            pltpu.make_async_copy(sd_h.at[b], SDV[c], sem.at[c]).wait()

        def rows_start(kq, c, stage_a):
            p = c & 1
            b = sid + kq * NSUB
            srcl = SDV[c].at[pl.ds(0, BLK)]
            pltpu.async_copy(ea_h.at[b], EAV[p], sem.at[8 + p])
            if stage_a:
                @pl.when(cid == 0)
                def _():
                    pltpu.async_copy(xwa_h.at[srcl], XWV[p], sem.at[10 + p])

                @pl.when(cid == 1)
                def _():
                    pltpu.async_copy(xwb_h.at[srcl], XWV[p], sem.at[10 + p])
            else:
                pltpu.async_copy(nf_sh.at[srcl], MFV[p], sem.at[10 + p])
            pltpu.async_copy(pos_h.at[SDV[c]], PSD[p], sem.at[12 + p])

        def rows_wait(kq, c, stage_a):
            p = c & 1
            b = sid + kq * NSUB
            srcl = SDV[c].at[pl.ds(0, BLK)]
            pltpu.make_async_copy(ea_h.at[b], EAV[p], sem.at[8 + p]).wait()
            if stage_a:
                @pl.when(cid == 0)
                def _():
                    pltpu.make_async_copy(
                        xwa_h.at[srcl], XWV[p], sem.at[10 + p]).wait()

                @pl.when(cid == 1)
                def _():
                    pltpu.make_async_copy(
                        xwb_h.at[srcl], XWV[p], sem.at[10 + p]).wait()
            else:
                pltpu.make_async_copy(
                    nf_sh.at[srcl], MFV[p], sem.at[10 + p]).wait()
            pltpu.make_async_copy(
                pos_h.at[SDV[c]], PSD[p], sem.at[12 + p]).wait()

        def scatter_start(c):
            p = c & 1
            pltpu.async_copy(MFV[p], nf_sh.at[SDV[c].at[pl.ds(BLK, BLK)]],
                             sem.at[16 + p], add=True)

        def scatter_wait(c):
            p = c & 1
            pltpu.make_async_copy(MFV[p],
                                  nf_sh.at[SDV[c].at[pl.ds(BLK, BLK)]],
                                  sem.at[16 + p]).wait()

        def sh_pass(p, want1, want2):
            for g in range(BLK // 16):
                r16 = iota16 + g * 16
                exv = (plsc.load_gather(PSD[p], [r16, ci32])
                       - plsc.load_gather(PSD[p], [r16 + BLK, ci32]))
                eyv = (plsc.load_gather(PSD[p], [r16, ci32 + 1])
                       - plsc.load_gather(PSD[p], [r16 + BLK, ci32 + 1]))
                ezv = (plsc.load_gather(PSD[p], [r16, ci32 + 2])
                       - plsc.load_gather(PSD[p], [r16 + BLK, ci32 + 2]))
                if want1:
                    sh_v[pl.ds(0 * BLK + g * 16, 16)] = S3 * exv
                    sh_v[pl.ds(1 * BLK + g * 16, 16)] = S3 * eyv
                    sh_v[pl.ds(2 * BLK + g * 16, 16)] = S3 * ezv
                if want2:
                    sh_v[pl.ds(3 * BLK + g * 16, 16)] = (S5 * S3) * (exv * ezv)
                    sh_v[pl.ds(4 * BLK + g * 16, 16)] = (S5 * S3) * (exv * eyv)
                    sh_v[pl.ds(5 * BLK + g * 16, 16)] = S5 * (
                        eyv * eyv - 0.5 * (exv * exv + ezv * ezv)
                    )
                    sh_v[pl.ds(6 * BLK + g * 16, 16)] = (S5 * S3) * (eyv * ezv)
                    sh_v[pl.ds(7 * BLK + g * 16, 16)] = (S5 * 0.5 * S3) * (
                        ezv * ezv - exv * exv
                    )

        def ea4(p, i):
            return [bcast(EAV[p], i * NB + v) for v in range(NB)]

        def q4(p, i, nj):
            e = ea4(p, i)
            q = []
            for j in range(nj):
                acc = XWV[p][i, pl.ds(0 * 64 + j * 16, 16)] * e[0]
                acc += XWV[p][i, pl.ds(1 * 64 + j * 16, 16)] * e[1]
                acc += XWV[p][i, pl.ds(2 * 64 + j * 16, 16)] * e[2]
                acc += XWV[p][i, pl.ds(3 * 64 + j * 16, 16)] * e[3]
                q.append(acc)
            return q

        def edge_a0(p):
            def edge(i, _):
                q = q4(p, i, 4)
                MFV[p][i, pl.ds(0, 16)] = q[0]
                MFV[p][i, pl.ds(16, 16)] = q[1]
                for m in range(3):
                    sm = bcast(sh_v, m * BLK + i)
                    MFV[p][i, pl.ds(32 + 24 * m, 16)] = q[2] * sm
                    MFV[p][i, pl.ds(48 + 24 * m, 16)] = q[3] * sm
                return 0

            lax.fori_loop(0, BLK, edge, 0)

        def edge_a1(p):
            def edge(i, _):
                q = q4(p, i, 3)
                MFV[p][i, pl.ds(0, 16)] = q[0]
                MFV[p][i, pl.ds(16, 16)] = q[1]
                for m in range(5):
                    sm = bcast(sh_v, (3 + m) * BLK + i)
                    MFV[p][i, pl.ds(32 + 16 * m, 16)] = q[2] * sm
                return 0

            lax.fori_loop(0, BLK, edge, 0)

        def a4(p, i, nj):
            e = ea4(p, i)
            a = []
            for j in range(nj):
                acc = w2_v[0, pl.ds(j * 16, 16)] * e[0]
                acc += w2_v[1, pl.ds(j * 16, 16)] * e[1]
                acc += w2_v[2, pl.ds(j * 16, 16)] * e[2]
                acc += w2_v[3, pl.ds(j * 16, 16)] * e[3]
                a.append(acc)
            return a

        def gupd(c, i, acc):
            d16 = bcast(SDV[c], BLK + i)
            g16 = plsc.load_gather(batch_v, [d16])
            plsc.addupdate_scatter(gacc_v, [g16 * 16 + iota16], acc)

        def edge_b0(p, c):
            def edge(i, _):
                a = a4(p, i, 4)
                acc = MFV[p][i, pl.ds(0, 16)] * a[0]
                acc += MFV[p][i, pl.ds(16, 16)] * a[1]
                for m in range(3):
                    sm = bcast(sh_v, m * BLK + i)
                    acc += (MFV[p][i, pl.ds(32 + 24 * m, 16)] * a[2]) * sm
                    acc += (MFV[p][i, pl.ds(48 + 24 * m, 16)] * a[3]) * sm
                gupd(c, i, acc)
                return 0

            lax.fori_loop(0, BLK, edge, 0)

        def edge_b1(p, c):
            def edge(i, _):
                a = a4(p, i, 3)
                acc = MFV[p][i, pl.ds(0, 16)] * a[0]
                acc += MFV[p][i, pl.ds(16, 16)] * a[1]
                for m in range(5):
                    sm = bcast(sh_v, (3 + m) * BLK + i)
                    acc += (MFV[p][i, pl.ds(32 + 16 * m, 16)] * a[2]) * sm
                gupd(c, i, acc)
                return 0

            lax.fori_loop(0, BLK, edge, 0)

        def block_a(kq, c, first, n1, n2):
            p = c & 1
            if n2:
                idx_start(kq + 2, (c + 2) % 4)
            rows_wait(kq, c, True)
            if n1:
                idx_wait(kq + 1, (c + 1) % 4)
                rows_start(kq + 1, (c + 1) % 4, True)

            @pl.when(cid == 0)
            def _():
                sh_pass(p, True, False)
                edge_a0(p)

            @pl.when(cid == 1)
            def _():
                sh_pass(p, False, True)
                edge_a1(p)

            if not first:
                scatter_wait((c - 1) % 4)
            scatter_start(c)

        def block_b(kq, c, n1, n2):
            p = c & 1
            if n2:
                idx_start(kq + 2, (c + 2) % 4)
            rows_wait(kq, c, False)
            if n1:
                idx_wait(kq + 1, (c + 1) % 4)
                rows_start(kq + 1, (c + 1) % 4, False)

            @pl.when(cid == 0)
            def _():
                sh_pass(p, True, False)
                edge_b0(p, c)

            @pl.when(cid == 1)
            def _():
                sh_pass(p, False, True)
                edge_b1(p, c)

        idx_start(0, 0)
        idx_start(1, 1)
        idx_wait(0, 0)
        rows_start(0, 0, True)
        block_a(0, 0, True, True, True)
        block_a(1, 1, False, True, True)
        block_a(2, 2, False, True, True)
        block_a(3, 3, False, True, True)

        def quad_a(q, _):
            k0 = q * 4
            block_a(k0, 0, False, True, True)
            block_a(k0 + 1, 1, False, True, True)
            block_a(k0 + 2, 2, False, True, True)
            block_a(k0 + 3, 3, False, True, True)
            return 0

        lax.fori_loop(1, NQ - 1, quad_a, 0)

        kl = (NQ - 1) * 4
        block_a(kl, 0, False, True, True)
        block_a(kl + 1, 1, False, True, True)
        block_a(kl + 2, 2, False, True, False)
        block_a(kl + 3, 3, False, False, False)
        scatter_wait(3)
        plsc.subcore_barrier()

        idx_start(0, 0)
        idx_start(1, 1)
        idx_wait(0, 0)
        rows_start(0, 0, False)
        block_b(0, 0, True, True)
        block_b(1, 1, True, True)
        block_b(2, 2, True, True)
        block_b(3, 3, True, True)

        def quad_b(q, _):
            k0 = q * 4
            block_b(k0, 0, True, True)
            block_b(k0 + 1, 1, True, True)
            block_b(k0 + 2, 2, True, True)
            block_b(k0 + 3, 3, True, True)
            return 0

        lax.fori_loop(1, NQ - 1, quad_b, 0)

        kl = (NQ - 1) * 4
        block_b(kl, 0, True, True)
        block_b(kl + 1, 1, True, True)
        block_b(kl + 2, 2, True, False)
        block_b(kl + 3, 3, False, False)

        pltpu.sync_copy(gacc_v, out_h.at[cid, sid])

    return body(xwa, xwb, pos, batch, sd_r, ea, w2t)


def kernel(positions, x, edge_attr, edge_index, batch, W1_0, W1_1, W1_2,
           W2_0, W2_1, W2_2):
    inv1 = 1.0 / float(np.sqrt(NA * NB))
    fan2 = 416.0
    k0 = 1.0 / float(np.sqrt(fan2))
    k1 = 1.0 / float(np.sqrt(fan2 * 3.0))
    k2 = 1.0 / float(np.sqrt(fan2 * 5.0))

    w1cat = jnp.concatenate([W1_0, W1_1, W1_2], axis=2) * inv1
    zpad8 = jnp.zeros((NA, 8), jnp.float32)
    zpad16 = jnp.zeros((NA, 16), jnp.float32)
    segs0 = [jnp.concatenate(
        [w1cat[:, v, 0:32], w1cat[:, v, 64:88], zpad8], axis=1)
        for v in range(NB)]
    segs1 = [jnp.concatenate(
        [w1cat[:, v, 32:64], w1cat[:, v, 88:104], zpad16], axis=1)
        for v in range(NB)]
    w1p0 = jnp.concatenate(segs0, axis=1)
    w1p1 = jnp.concatenate(segs1, axis=1)
    w1p = jnp.stack([w1p0, w1p1])

    z8 = jnp.zeros((NB, 8), jnp.float32)
    z16 = jnp.zeros((NB, 16), jnp.float32)
    w2t0 = jnp.concatenate(
        [W2_0[0:32, :, 0].T * k0, W2_1[:, :, 0].T * k1, z8], axis=1)
    w2t1 = jnp.concatenate(
        [W2_0[32:64, :, 0].T * k0, W2_2[:, :, 0].T * k2, z16], axis=1)
    w2t = jnp.stack([w2t0, w2t1])

    xp = jnp.concatenate(
        [x, jnp.zeros((NP - N, NA), jnp.float32)], axis=0)
    xw = _k1_xw(xp, w1p)

    pad_i = jnp.full((EP - E,), N, edge_index.dtype)
    src_r = jnp.concatenate([edge_index[0], pad_i]).reshape(NBLKS, BLK)
    dst_r = jnp.concatenate([edge_index[1], pad_i]).reshape(NBLKS, BLK)
    sd_r = jnp.concatenate(
        [src_r[:, None, :], dst_r[:, None, :]], axis=1
    ).reshape(NBLKS, 2 * BLK)
    ea_r = jnp.concatenate(
        [edge_attr, jnp.zeros((EP - E, NB), edge_attr.dtype)]
    ).reshape(NBLKS, BLK * NB)
    batch_p = jnp.concatenate(
        [batch, jnp.zeros((NP - N,), batch.dtype)])
    posp = jnp.pad(positions, ((0, NP - N), (0, 13)))

    partials = _sc_kernel(xw[0], xw[1], posp, batch_p, sd_r, ea_r, w2t)
    return _k3_reduce(partials.reshape(NCORE, NSUB, G, 16))
